# baseline (device time: 31876 ns/iter reference)
import jax
import jax.numpy as jnp
from jax import lax
from jax.experimental import pallas as pl
from jax.experimental.pallas import tpu as pltpu

N_LAYERS = 3
NC = 4


def kernel(x, Win0, Wout0, Win1, Wout1, Win2, Wout2):
    b, d_loc = x.shape
    d_in, h_loc = Win0.shape
    bc = b // NC

    def rows(c):
        return pl.ds(c * bc, bc)

    def body(x_ref, win0_ref, wout0_ref, win1_ref, wout1_ref, win2_ref,
             wout2_ref, out_ref,
             winb_ref, woutb_ref,
             winp_ref, woutp_ref,
             hbuf_ref,
             ysend_ref, ybuf_ref,
             usend_ref, urecv_ref,
             vsend_ref, vrecv_ref,
             ysend_sem, yrecv_sem,
             wsend_sem, wrecv_sem,
             usend_sem, urecv_sem,
             vsend_sem, vrecv_sem):
        my_x = lax.axis_index("x")
        my_y = lax.axis_index("y")
        y_peer = (my_x, 1 - my_y)
        x_peer = (1 - my_x, my_y)

        barrier_sem = pltpu.get_barrier_semaphore()
        for nbr in (y_peer, x_peer):
            pl.semaphore_signal(
                barrier_sem, inc=1,
                device_id=nbr, device_id_type=pl.DeviceIdType.MESH,
            )

        wins = (win0_ref, win1_ref, win2_ref)
        wouts = (wout0_ref, wout1_ref, wout2_ref)
        for l in range(N_LAYERS):
            winb_ref[l, :, :] = wins[l][:, :].astype(jnp.bfloat16)
            woutb_ref[l, :, :] = wouts[l][:, :].astype(jnp.bfloat16)

        pl.semaphore_wait(barrier_sem, 2)

        def y_rdma(c):
            return pltpu.make_async_remote_copy(
                src_ref=ysend_ref.at[rows(c)], dst_ref=ybuf_ref.at[rows(c)],
                send_sem=ysend_sem.at[c], recv_sem=yrecv_sem.at[c],
                device_id=y_peer, device_id_type=pl.DeviceIdType.MESH)

        _w_src = (woutb_ref.at[0], winb_ref.at[1], woutb_ref.at[1],
                  winb_ref.at[2])
        _w_dst = (woutp_ref.at[0], winp_ref.at[0], woutp_ref.at[1],
                  winp_ref.at[1])

        def w_rdma(k):
            return pltpu.make_async_remote_copy(
                src_ref=_w_src[k], dst_ref=_w_dst[k],
                send_sem=wsend_sem.at[k], recv_sem=wrecv_sem.at[k],
                device_id=y_peer, device_id_type=pl.DeviceIdType.MESH)

        def u_rdma(l, c):
            return pltpu.make_async_remote_copy(
                src_ref=usend_ref.at[l, rows(c)],
                dst_ref=urecv_ref.at[l, rows(c)],
                send_sem=usend_sem.at[l, c], recv_sem=urecv_sem.at[l, c],
                device_id=x_peer, device_id_type=pl.DeviceIdType.MESH)

        def v_rdma(c):
            return pltpu.make_async_remote_copy(
                src_ref=vsend_ref.at[rows(c)], dst_ref=vrecv_ref.at[rows(c)],
                send_sem=vsend_sem.at[c], recv_sem=vrecv_sem.at[c],
                device_id=x_peer, device_id_type=pl.DeviceIdType.MESH)

        act = x_ref[:, :].astype(jnp.bfloat16)
        for c in range(NC):
            p1 = jnp.dot(act[c * bc:(c + 1) * bc, :], winb_ref[0],
                         preferred_element_type=jnp.float32)
            ysend_ref[rows(c)] = p1.astype(jnp.bfloat16)
            y_rdma(c).start()
        for k in range(4):
            w_rdma(k).start()

        for c in range(NC):
            y_rdma(c).wait_recv()
            h0 = jnp.maximum(ysend_ref[rows(c)].astype(jnp.float32)
                             + ybuf_ref[rows(c)].astype(jnp.float32),
                             0.0).astype(jnp.bfloat16)
            hbuf_ref[rows(c)] = h0
            uo = jnp.dot(h0, woutb_ref[0], preferred_element_type=jnp.float32)
            usend_ref[0, rows(c), 0:d_loc] = uo.astype(jnp.bfloat16)
        w_rdma(0).wait_recv()
        for c in range(NC):
            up = jnp.dot(hbuf_ref[rows(c)], woutp_ref[0],
                         preferred_element_type=jnp.float32)
            usend_ref[0, rows(c), d_loc:2 * d_loc] = up.astype(jnp.bfloat16)
            u_rdma(0, c).start()

        w_rdma(1).wait_recv()
        for c in range(NC):
            u_rdma(0, c).wait_recv()
            x1 = (usend_ref[0, rows(c)].astype(jnp.float32)
                  + urecv_ref[0, rows(c)].astype(jnp.float32)
                  ).astype(jnp.bfloat16)
            t1 = (jnp.dot(x1[:, 0:d_loc], winb_ref[1],
                          preferred_element_type=jnp.float32)
                  + jnp.dot(x1[:, d_loc:2 * d_loc], winp_ref[0],
                            preferred_element_type=jnp.float32))
            h1 = jnp.maximum(t1, 0.0).astype(jnp.bfloat16)
            uo = jnp.dot(h1, woutb_ref[1], preferred_element_type=jnp.float32)
            usend_ref[1, rows(c), 0:d_loc] = uo.astype(jnp.bfloat16)
            if c == 0:
                w_rdma(2).wait_recv()
            up = jnp.dot(h1, woutp_ref[1], preferred_element_type=jnp.float32)
            usend_ref[1, rows(c), d_loc:2 * d_loc] = up.astype(jnp.bfloat16)
            u_rdma(1, c).start()

        for c in range(NC):
            u_rdma(1, c).wait_recv()
            x2 = (usend_ref[1, rows(c)].astype(jnp.float32)
                  + urecv_ref[1, rows(c)].astype(jnp.float32)
                  ).astype(jnp.bfloat16)
            t2o = jnp.dot(x2[:, 0:d_loc], winb_ref[2],
                          preferred_element_type=jnp.float32)
            if c == 0:
                w_rdma(3).wait_recv()
            t2 = t2o + jnp.dot(x2[:, d_loc:2 * d_loc], winp_ref[1],
                               preferred_element_type=jnp.float32)
            h2 = jnp.maximum(t2, 0.0).astype(jnp.bfloat16)
            v = jnp.dot(h2, woutb_ref[2], preferred_element_type=jnp.float32)
            vsend_ref[rows(c)] = v.astype(jnp.bfloat16)
            v_rdma(c).start()

        for c in range(NC):
            v_rdma(c).wait_recv()
            out_ref[rows(c)] = (vsend_ref[rows(c)].astype(jnp.float32)
                                + vrecv_ref[rows(c)].astype(jnp.float32))

        for c in range(NC):
            y_rdma(c).wait_send()
            u_rdma(0, c).wait_send()
            u_rdma(1, c).wait_send()
            v_rdma(c).wait_send()
        for k in range(4):
            w_rdma(k).wait_send()

    return pl.pallas_call(
        body,
        out_shape=jax.ShapeDtypeStruct((b, d_loc), jnp.float32),
        in_specs=[pl.BlockSpec(memory_space=pltpu.VMEM)] * 7,
        out_specs=pl.BlockSpec(memory_space=pltpu.VMEM),
        scratch_shapes=[
            pltpu.VMEM((N_LAYERS, d_in, h_loc), jnp.bfloat16),
            pltpu.VMEM((N_LAYERS, h_loc, d_loc), jnp.bfloat16),
            pltpu.VMEM((2, d_in, h_loc), jnp.bfloat16),
            pltpu.VMEM((2, h_loc, d_loc), jnp.bfloat16),
            pltpu.VMEM((b, h_loc), jnp.bfloat16),
            pltpu.VMEM((b, h_loc), jnp.bfloat16),
            pltpu.VMEM((b, h_loc), jnp.bfloat16),
            pltpu.VMEM((2, b, h_loc), jnp.bfloat16),
            pltpu.VMEM((2, b, h_loc), jnp.bfloat16),
            pltpu.VMEM((b, d_loc), jnp.bfloat16),
            pltpu.VMEM((b, d_loc), jnp.bfloat16),
            pltpu.SemaphoreType.DMA((NC,)),
            pltpu.SemaphoreType.DMA((NC,)),
            pltpu.SemaphoreType.DMA((4,)),
            pltpu.SemaphoreType.DMA((4,)),
            pltpu.SemaphoreType.DMA((2, NC)),
            pltpu.SemaphoreType.DMA((2, NC)),
            pltpu.SemaphoreType.DMA((NC,)),
            pltpu.SemaphoreType.DMA((NC,)),
        ],
        compiler_params=pltpu.CompilerParams(collective_id=0),
    )(x, Win0, Wout0, Win1, Wout1, Win2, Wout2)


# device time: 28289 ns/iter; 1.1268x vs baseline; 1.1268x over previous
import jax
import jax.numpy as jnp
from jax import lax
from jax.experimental import pallas as pl
from jax.experimental.pallas import tpu as pltpu

N_LAYERS = 3
NC = 4


def kernel(x, Win0, Wout0, Win1, Wout1, Win2, Wout2):
    b, d_loc = x.shape
    d_in, h_loc = Win0.shape
    bc = b // NC

    def rows(c):
        return pl.ds(c * bc, bc)

    def body(x_ref, win0_ref, wout0_ref, win1_ref, wout1_ref, win2_ref,
             wout2_ref, out_ref, winb_ref, woutb_ref, ysend_ref, xsend_ref,
             ybuf_ref, xbuf_ref, ysend_sem, yrecv_sem, xsend_sem, xrecv_sem):
        my_x = lax.axis_index("x")
        my_y = lax.axis_index("y")
        y_peer = (my_x, 1 - my_y)
        x_peer = (1 - my_x, my_y)

        barrier_sem = pltpu.get_barrier_semaphore()
        for nbr in (y_peer, x_peer):
            pl.semaphore_signal(
                barrier_sem, inc=1,
                device_id=nbr, device_id_type=pl.DeviceIdType.MESH,
            )

        wins = (win0_ref, win1_ref, win2_ref)
        wouts = (wout0_ref, wout1_ref, wout2_ref)
        for l in range(N_LAYERS):
            winb_ref[l, :, :] = wins[l][:, :].astype(jnp.bfloat16)
            woutb_ref[l, :, :] = wouts[l][:, :].astype(jnp.bfloat16)

        pl.semaphore_wait(barrier_sem, 2)

        def y_rdma(l, c):
            return pltpu.make_async_remote_copy(
                src_ref=ysend_ref.at[l, rows(c)],
                dst_ref=ybuf_ref.at[l, rows(c)],
                send_sem=ysend_sem.at[l, c],
                recv_sem=yrecv_sem.at[l, c],
                device_id=y_peer,
                device_id_type=pl.DeviceIdType.MESH,
            )

        def x_rdma(l, c):
            return pltpu.make_async_remote_copy(
                src_ref=xsend_ref.at[l, rows(c)],
                dst_ref=xbuf_ref.at[l, rows(c)],
                send_sem=xsend_sem.at[l, c],
                recv_sem=xrecv_sem.at[l, c],
                device_id=x_peer,
                device_id_type=pl.DeviceIdType.MESH,
            )

        act = x_ref[:, :].astype(jnp.bfloat16)
        for c in range(NC):
            p1 = jnp.dot(act[c * bc:(c + 1) * bc, :], winb_ref[0],
                         preferred_element_type=jnp.float32)
            ysend_ref[0, rows(c)] = p1.astype(jnp.bfloat16)
            y_rdma(0, c).start()

        for l in range(N_LAYERS):
            for c in range(NC):
                y_rdma(l, c).wait_recv()
                hsum = (ysend_ref[l, rows(c)].astype(jnp.float32)
                        + ybuf_ref[l, rows(c)].astype(jnp.float32))
                hb = jnp.maximum(hsum, 0.0).astype(jnp.bfloat16)
                p2 = jnp.dot(hb, woutb_ref[l],
                             preferred_element_type=jnp.float32)
                xsend_ref[l, rows(c)] = p2.astype(jnp.bfloat16)
                x_rdma(l, c).start()
            for c in range(NC):
                x_rdma(l, c).wait_recv()
                asum = (xsend_ref[l, rows(c)].astype(jnp.float32)
                        + xbuf_ref[l, rows(c)].astype(jnp.float32))
                if l == N_LAYERS - 1:
                    out_ref[rows(c), :] = asum
                else:
                    p1 = jnp.dot(asum.astype(jnp.bfloat16), winb_ref[l + 1],
                                 preferred_element_type=jnp.float32)
                    ysend_ref[l + 1, rows(c)] = p1.astype(jnp.bfloat16)
                    y_rdma(l + 1, c).start()

        for l in range(N_LAYERS):
            for c in range(NC):
                y_rdma(l, c).wait_send()
                x_rdma(l, c).wait_send()

    return pl.pallas_call(
        body,
        out_shape=jax.ShapeDtypeStruct((b, d_loc), jnp.float32),
        in_specs=[pl.BlockSpec(memory_space=pltpu.VMEM)] * 7,
        out_specs=pl.BlockSpec(memory_space=pltpu.VMEM),
        scratch_shapes=[
            pltpu.VMEM((N_LAYERS, d_in, h_loc), jnp.bfloat16),
            pltpu.VMEM((N_LAYERS, h_loc, d_loc), jnp.bfloat16),
            pltpu.VMEM((N_LAYERS, b, h_loc), jnp.bfloat16),
            pltpu.VMEM((N_LAYERS, b, d_loc), jnp.bfloat16),
            pltpu.VMEM((N_LAYERS, b, h_loc), jnp.bfloat16),
            pltpu.VMEM((N_LAYERS, b, d_loc), jnp.bfloat16),
            pltpu.SemaphoreType.DMA((N_LAYERS, NC)),
            pltpu.SemaphoreType.DMA((N_LAYERS, NC)),
            pltpu.SemaphoreType.DMA((N_LAYERS, NC)),
            pltpu.SemaphoreType.DMA((N_LAYERS, NC)),
        ],
        compiler_params=pltpu.CompilerParams(collective_id=0),
    )(x, Win0, Wout0, Win1, Wout1, Win2, Wout2)
